# Initial kernel scaffold; baseline (speedup 1.0000x reference)
import jax
import jax.numpy as jnp
from jax.experimental import pallas as pl


def kernel(hidden_states, Wq, Wk, Wv, Wo):
    raise NotImplementedError("write your pallas kernel here")


# speedup vs baseline: 3.0177x; 3.0177x over previous
import functools

import jax
import jax.numpy as jnp
import numpy as np
from jax.experimental import pallas as pl

S = 2048
D_MODEL = 1024
NUM_HEADS = 16
NUM_KV_HEADS = 4
HEAD_DIM = 64
N_REP = NUM_HEADS // NUM_KV_HEADS
KV_D = NUM_KV_HEADS * HEAD_DIM
_PID = (0, 1, 2, 3, 50, 100)
_LAYER_IDX = 8
_KK = int(192 - _LAYER_IDX / 31 * (192 - 64))

_HI = jax.lax.Precision.HIGHEST


def _kv_proj_kernel(x_ref, w_ref, o_ref):
    o_ref[...] = jnp.dot(x_ref[...], w_ref[...], precision=_HI)


def _attn_kernel(xl_ref, wq_ref, k_ref, v_ref, prow_ref, ctx_ref):
    q = jnp.dot(xl_ref[...], wq_ref[...], precision=_HI)
    k = k_ref[...]
    rows = []
    for h in range(NUM_HEADS):
        qh = q[:, h * HEAD_DIM : (h + 1) * HEAD_DIM]
        g = h // N_REP
        kg = k[:, g * HEAD_DIM : (g + 1) * HEAD_DIM]
        rows.append(
            jax.lax.dot_general(
                qh, kg, (((1,), (1,)), ((), ())), precision=_HI
            )
        )
    scores = jnp.concatenate(rows, axis=0) * (
        1.0 / np.sqrt(HEAD_DIM)
    )
    m = jnp.max(scores, axis=1, keepdims=True)
    e = jnp.exp(scores - m)
    attn = e / jnp.sum(e, axis=1, keepdims=True)

    imp = jnp.mean(attn, axis=0, keepdims=True)
    lane = jax.lax.broadcasted_iota(jnp.int32, (1, S), 1)
    is_pid = functools.reduce(
        jnp.logical_or, [lane == p for p in _PID]
    )
    imp = jnp.where(is_pid, imp * 2.5, imp)

    bits = jax.lax.bitcast_convert_type(imp, jnp.int32)

    def body(i, t):
        cand = t | jax.lax.shift_left(jnp.int32(1), 30 - i)
        cnt = jnp.sum((bits >= cand).astype(jnp.int32))
        return jnp.where(cnt >= _KK, cand, t)

    t = jax.lax.fori_loop(0, 31, body, jnp.int32(0))

    gt = bits > t
    eq = bits == t
    need = (_KK - jnp.sum(gt.astype(jnp.int32))).astype(jnp.float32)
    c = eq.astype(jnp.float32)
    sh = 1
    while sh < S:
        c = c + jnp.concatenate(
            [jnp.zeros((1, sh), c.dtype), c[:, : S - sh]], axis=1
        )
        sh *= 2
    sel = jnp.logical_or(gt, jnp.logical_and(eq, c <= need))

    prow = attn * sel.astype(jnp.float32)
    prow_ref[...] = prow

    ctx = jnp.dot(prow, v_ref[...], precision=_HI)
    hh = jax.lax.broadcasted_iota(jnp.int32, (NUM_HEADS, KV_D), 0)
    gg = jax.lax.broadcasted_iota(jnp.int32, (NUM_HEADS, KV_D), 1) // HEAD_DIM
    ctx = jnp.where(hh // N_REP == gg, ctx, 0.0)
    ctx_ref[...] = (
        ctx[:, 0:64] + ctx[:, 64:128] + ctx[:, 128:192] + ctx[:, 192:256]
    )


def _prune_fill_kernel(prow_ref, o_ref, *, nj, rows):
    j = pl.program_id(1)
    o_ref[...] = jnp.zeros_like(o_ref)

    @pl.when(j == nj - 1)
    def _():
        o_ref[:, rows - 1, :] = prow_ref[:, 0, :]


def _out_fill_kernel(ctx_ref, wo_ref, o_ref, *, nj, rows):
    j = pl.program_id(0)
    o_ref[...] = jnp.zeros_like(o_ref)

    @pl.when(j == nj - 1)
    def _():
        o_ref[rows - 1 : rows, :] = jnp.dot(
            ctx_ref[...], wo_ref[...], precision=_HI
        )


def kernel(hidden_states, Wq, Wk, Wv, Wo):
    x = hidden_states[0]
    Wkv = jnp.concatenate([Wk, Wv], axis=1)

    bs1 = 256
    kv = pl.pallas_call(
        _kv_proj_kernel,
        grid=(S // bs1,),
        in_specs=[
            pl.BlockSpec((bs1, D_MODEL), lambda i: (i, 0)),
            pl.BlockSpec((D_MODEL, 2 * KV_D), lambda i: (0, 0)),
        ],
        out_specs=pl.BlockSpec((bs1, 2 * KV_D), lambda i: (i, 0)),
        out_shape=jax.ShapeDtypeStruct((S, 2 * KV_D), jnp.float32),
    )(x, Wkv)
    k_flat = kv[:, :KV_D]
    v_flat = kv[:, KV_D:]

    x_last = x[S - 1 :, :]
    prow, ctx16 = pl.pallas_call(
        _attn_kernel,
        out_shape=(
            jax.ShapeDtypeStruct((NUM_HEADS, S), jnp.float32),
            jax.ShapeDtypeStruct((NUM_HEADS, HEAD_DIM), jnp.float32),
        ),
    )(x_last, Wq, k_flat, v_flat)
    ctx = ctx16.reshape(1, NUM_HEADS * HEAD_DIM)

    rows = 256
    nj = S // rows
    pruned = pl.pallas_call(
        functools.partial(_prune_fill_kernel, nj=nj, rows=rows),
        grid=(NUM_HEADS, nj),
        in_specs=[pl.BlockSpec((1, 1, S), lambda h, j: (h, 0, 0))],
        out_specs=pl.BlockSpec((1, rows, S), lambda h, j: (h, j, 0)),
        out_shape=jax.ShapeDtypeStruct((NUM_HEADS, S, S), jnp.float32),
    )(prow.reshape(NUM_HEADS, 1, S))

    orows = 256
    onj = S // orows
    out = pl.pallas_call(
        functools.partial(_out_fill_kernel, nj=onj, rows=orows),
        grid=(onj,),
        in_specs=[
            pl.BlockSpec((1, NUM_HEADS * HEAD_DIM), lambda j: (0, 0)),
            pl.BlockSpec((NUM_HEADS * HEAD_DIM, D_MODEL), lambda j: (0, 0)),
        ],
        out_specs=pl.BlockSpec((orows, D_MODEL), lambda j: (j, 0)),
        out_shape=jax.ShapeDtypeStruct((S, D_MODEL), jnp.float32),
    )(ctx, Wo)

    k_kv = k_flat.reshape(1, S, NUM_KV_HEADS, HEAD_DIM).transpose(0, 2, 1, 3)
    v_kv = v_flat.reshape(1, S, NUM_KV_HEADS, HEAD_DIM).transpose(0, 2, 1, 3)
    return out[None], pruned[None], k_kv, v_kv


# speedup vs baseline: 4.6939x; 1.5555x over previous
import functools

import jax
import jax.numpy as jnp
import numpy as np
from jax.experimental import pallas as pl
from jax.experimental.pallas import tpu as pltpu

S = 2048
D_MODEL = 1024
NUM_HEADS = 16
NUM_KV_HEADS = 4
HEAD_DIM = 64
N_REP = NUM_HEADS // NUM_KV_HEADS
KV_D = NUM_KV_HEADS * HEAD_DIM
_PID = (0, 1, 2, 3, 50, 100)
_LAYER_IDX = 8
_KK = int(192 - _LAYER_IDX / 31 * (192 - 64))

_HI = jax.lax.Precision.HIGHEST
_ZROWS = 512


def _mega_kernel(
    x_ref, wq_ref, wkv_ref, wo_ref,
    pruned_ref, out_ref, kv_ref,
    zbuf, prow_buf, olast_buf, sems,
):
    zbuf[...] = jnp.zeros_like(zbuf)
    copies = []
    n = 0
    nblk = (S - 8) // _ZROWS
    rem = (S - 8) - nblk * _ZROWS
    for h in range(NUM_HEADS):
        for j in range(nblk):
            c = pltpu.make_async_copy(
                zbuf,
                pruned_ref.at[h, j * _ZROWS : (j + 1) * _ZROWS, :],
                sems.at[n],
            )
            c.start()
            copies.append(c)
            n += 1
        c = pltpu.make_async_copy(
            zbuf.at[0:rem, :],
            pruned_ref.at[h, nblk * _ZROWS : S - 8, :],
            sems.at[n],
        )
        c.start()
        copies.append(c)
        n += 1
    for j in range(nblk):
        c = pltpu.make_async_copy(
            zbuf.at[:, 0:D_MODEL],
            out_ref.at[j * _ZROWS : (j + 1) * _ZROWS, :],
            sems.at[n],
        )
        c.start()
        copies.append(c)
        n += 1
    c = pltpu.make_async_copy(
        zbuf.at[0:rem, 0:D_MODEL],
        out_ref.at[nblk * _ZROWS : S - 8, :],
        sems.at[n],
    )
    c.start()
    copies.append(c)
    n += 1

    half = S // 2
    for i in range(2):
        kv_ref[i * half : (i + 1) * half, :] = jnp.dot(
            x_ref[i * half : (i + 1) * half, :].astype(jnp.bfloat16),
            wkv_ref[...].astype(jnp.bfloat16),
            preferred_element_type=jnp.float32,
        )
    k = kv_ref[:, :KV_D]
    v = kv_ref[:, KV_D:]

    q = jnp.dot(
        x_ref[S - 1 : S, :].astype(jnp.bfloat16),
        wq_ref[...].astype(jnp.bfloat16),
        preferred_element_type=jnp.float32,
    )
    rows = []
    for h in range(NUM_HEADS):
        qh = q[:, h * HEAD_DIM : (h + 1) * HEAD_DIM].astype(jnp.bfloat16)
        g = h // N_REP
        kg = k[:, g * HEAD_DIM : (g + 1) * HEAD_DIM].astype(jnp.bfloat16)
        rows.append(
            jax.lax.dot_general(
                qh, kg, (((1,), (1,)), ((), ())),
                preferred_element_type=jnp.float32,
            )
        )
    scores = jnp.concatenate(rows, axis=0) * (
        1.0 / np.sqrt(HEAD_DIM)
    )
    m = jnp.max(scores, axis=1, keepdims=True)
    e = jnp.exp(scores - m)
    attn = e / jnp.sum(e, axis=1, keepdims=True)

    imp = jnp.mean(attn, axis=0, keepdims=True)
    lane = jax.lax.broadcasted_iota(jnp.int32, (1, S), 1)
    is_pid = functools.reduce(jnp.logical_or, [lane == p for p in _PID])
    imp = jnp.where(is_pid, imp * 2.5, imp)

    bits = jax.lax.bitcast_convert_type(imp, jnp.int32)

    def body(i, t):
        cand = t | jax.lax.shift_left(jnp.int32(1), 30 - i)
        cnt = jnp.sum((bits >= cand).astype(jnp.int32))
        return jnp.where(cnt >= _KK, cand, t)

    t = jax.lax.fori_loop(0, 31, body, jnp.int32(0))

    gt = bits > t
    eq = bits == t
    need = (_KK - jnp.sum(gt.astype(jnp.int32))).astype(jnp.float32)
    c32 = eq.astype(jnp.float32)
    sh = 1
    while sh < S:
        c32 = c32 + jnp.concatenate(
            [jnp.zeros((1, sh), c32.dtype), c32[:, : S - sh]], axis=1
        )
        sh *= 2
    sel = jnp.logical_or(gt, jnp.logical_and(eq, c32 <= need))

    prow = attn * sel.astype(jnp.float32)
    prow_buf[...] = jnp.zeros_like(prow_buf)
    for h in range(NUM_HEADS):
        prow_buf[8 * h + 7 : 8 * h + 8, :] = prow[h : h + 1, :]

    ctx = jnp.dot(
        prow.astype(jnp.bfloat16),
        v.astype(jnp.bfloat16),
        preferred_element_type=jnp.float32,
    )
    hh = jax.lax.broadcasted_iota(jnp.int32, (NUM_HEADS, KV_D), 0)
    gg = jax.lax.broadcasted_iota(jnp.int32, (NUM_HEADS, KV_D), 1) // HEAD_DIM
    ctx = jnp.where(hh // N_REP == gg, ctx, 0.0)
    ctx16 = (
        ctx[:, 0:64] + ctx[:, 64:128] + ctx[:, 128:192] + ctx[:, 192:256]
    )

    olast = jnp.zeros((1, D_MODEL), jnp.float32)
    for h in range(NUM_HEADS):
        olast = olast + jnp.dot(
            ctx16[h : h + 1, :].astype(jnp.bfloat16),
            wo_ref[h * HEAD_DIM : (h + 1) * HEAD_DIM, :].astype(jnp.bfloat16),
            preferred_element_type=jnp.float32,
        )
    olast_buf[...] = jnp.zeros_like(olast_buf)
    olast_buf[7:8, :] = olast

    for h in range(NUM_HEADS):
        c = pltpu.make_async_copy(
            prow_buf.at[8 * h : 8 * (h + 1), :],
            pruned_ref.at[h, S - 8 : S, :],
            sems.at[n],
        )
        c.start()
        copies.append(c)
        n += 1
    c = pltpu.make_async_copy(olast_buf, out_ref.at[S - 8 : S, :], sems.at[n])
    c.start()
    copies.append(c)
    n += 1

    for c in copies:
        c.wait()


def kernel(hidden_states, Wq, Wk, Wv, Wo):
    x = hidden_states[0]
    Wkv = jnp.concatenate([Wk, Wv], axis=1)

    nsem = 2 * NUM_HEADS + 2 + NUM_HEADS + 1 + 1
    pruned, out, kv = pl.pallas_call(
        _mega_kernel,
        in_specs=[
            pl.BlockSpec(memory_space=pltpu.MemorySpace.VMEM),
            pl.BlockSpec(memory_space=pltpu.MemorySpace.VMEM),
            pl.BlockSpec(memory_space=pltpu.MemorySpace.VMEM),
            pl.BlockSpec(memory_space=pltpu.MemorySpace.VMEM),
        ],
        out_specs=(
            pl.BlockSpec(memory_space=pltpu.MemorySpace.HBM),
            pl.BlockSpec(memory_space=pltpu.MemorySpace.HBM),
            pl.BlockSpec(memory_space=pltpu.MemorySpace.VMEM),
        ),
        out_shape=(
            jax.ShapeDtypeStruct((NUM_HEADS, S, S), jnp.float32),
            jax.ShapeDtypeStruct((S, D_MODEL), jnp.float32),
            jax.ShapeDtypeStruct((S, 2 * KV_D), jnp.float32),
        ),
        scratch_shapes=[
            pltpu.VMEM((_ZROWS, S), jnp.float32),
            pltpu.VMEM((8 * NUM_HEADS, S), jnp.float32),
            pltpu.VMEM((8, D_MODEL), jnp.float32),
            pltpu.SemaphoreType.DMA((128,)),
        ],
    )(x, Wq, Wkv, Wo)

    k_flat = kv[:, :KV_D]
    v_flat = kv[:, KV_D:]
    k_kv = k_flat.reshape(1, S, NUM_KV_HEADS, HEAD_DIM).transpose(0, 2, 1, 3)
    v_kv = v_flat.reshape(1, S, NUM_KV_HEADS, HEAD_DIM).transpose(0, 2, 1, 3)
    return out[None], pruned[None], k_kv, v_kv
